# baseline (speedup 1.0000x reference)
import functools

import jax
import jax.numpy as jnp
from jax.experimental import pallas as pl
from jax.experimental.pallas import tpu as pltpu

NUM_STEPS = 100
K = 16
BIG = 1e30
SELF_BIG = 2e30


def _fused_kernel(x_ref, e_ref, aux_ref, w1x_ref, w1m_ref, w1c2_ref, w1c4_ref,
                  b1_ref, w2_ref, b2_ref, out_ref, scr_ref, *, R, N):
    b = pl.program_id(0)
    rb = pl.program_id(1)

    @pl.when(jnp.logical_and(b == 0, rb == 0))
    def _init():
        out_ref[...] = jnp.zeros((1, 1), jnp.float32)

    x_all = x_ref[0]
    e_all = e_ref[0]
    c0 = aux_ref[0, 0, 128]
    c1 = aux_ref[0, 0, 129]

    xn_all = c0 * x_all + c1 * e_all

    S = 16

    def strip(i, _):
        r0 = i * S
        xr = x_ref[0, pl.ds(rb * R + r0, S), :]
        dx = xr[:, 0:1] - x_all[:, 0].reshape(1, N)
        dy = xr[:, 1:2] - x_all[:, 1].reshape(1, N)
        d2 = dx * dx + dy * dy

        col = jax.lax.broadcasted_iota(jnp.int32, (S, N), 1)
        row = jax.lax.broadcasted_iota(jnp.int32, (S, N), 0) + (rb * R + r0)
        bits = jax.lax.bitcast_convert_type(d2, jnp.uint32)
        packed_bits = (bits & jnp.uint32(0xFFFFF800)) | col.astype(jnp.uint32)
        p = jax.lax.bitcast_convert_type(packed_bits, jnp.float32)
        p = jnp.where(col == row, SELF_BIG, p)

        for _ in range(K):
            m = jnp.min(p, axis=1, keepdims=True)
            p = jnp.where(p == m, BIG, p)
        scr_ref[pl.ds(r0, S), :] = (p == BIG).astype(jnp.float32)
        return 0

    jax.lax.fori_loop(0, R // S, strip, 0)
    sel = scr_ref[...]
    acc = jax.lax.dot_general(sel, xn_all, (((1,), (0,)), ((), ())),
                              preferred_element_type=jnp.float32)
    xm = acc * jnp.float32(1.0 / K)

    ctxfeat = aux_ref[0, :, 0:128]
    w1c = w1c2_ref[...] + w1c4_ref[...]
    bias = jax.lax.dot_general(ctxfeat, w1c, (((1,), (0,)), ((), ())),
                               preferred_element_type=jnp.float32) + b1_ref[...]

    e_r = e_ref[0, pl.ds(rb * R, R), :]
    x_r = x_ref[0, pl.ds(rb * R, R), :]
    xn_r = c0 * x_r + c1 * e_r
    pre = (jax.lax.dot_general(xn_r, w1x_ref[...], (((1,), (0,)), ((), ())),
                               preferred_element_type=jnp.float32)
           + jax.lax.dot_general(xm, w1m_ref[...], (((1,), (0,)), ((), ())),
                                 preferred_element_type=jnp.float32)
           + bias)
    h = jnp.maximum(pre, 0.0)
    out = jax.lax.dot_general(h, w2_ref[...], (((1,), (0,)), ((), ())),
                              preferred_element_type=jnp.float32) + b2_ref[...]
    err = out - e_r
    out_ref[...] += jnp.sum(err * err, keepdims=True)


def kernel(x_0, context, t, k, e_rand, W1, b1, W2, b2):
    B, N, D = x_0.shape
    R = min(1024, N)
    NB = N // R

    betas = jnp.linspace(1e-4, 0.02, NUM_STEPS).astype(jnp.float32)
    alphas = 1.0 - betas
    alpha_bars = jnp.cumprod(alphas)
    alpha_bar = alpha_bars[t]
    beta = betas[t]
    c0 = jnp.sqrt(alpha_bar)
    c1 = jnp.sqrt(1.0 - alpha_bar)

    aux = jnp.concatenate([
        context,
        beta[:, None], jnp.sin(beta)[:, None], jnp.cos(beta)[:, None],
        c0[:, None], c1[:, None],
        jnp.zeros((B, 126), jnp.float32),
    ], axis=1)

    CIN = context.shape[1] + 3 + D
    w1x = W1[0:D]
    w1m = W1[CIN:CIN + D]
    w1c2 = W1[D:CIN]
    w1c4 = W1[CIN + D:2 * CIN]
    HID = W1.shape[1]

    grid = (B, NB)
    full = lambda bi, ri: (0, 0)
    out = pl.pallas_call(
        functools.partial(_fused_kernel, R=R, N=N),
        grid=grid,
        in_specs=[
            pl.BlockSpec((1, N, D), lambda bi, ri: (bi, 0, 0)),
            pl.BlockSpec((1, N, D), lambda bi, ri: (bi, 0, 0)),
            pl.BlockSpec((1, 1, 256), lambda bi, ri: (bi, 0, 0)),
            pl.BlockSpec((D, HID), full),
            pl.BlockSpec((D, HID), full),
            pl.BlockSpec((CIN - D, HID), full),
            pl.BlockSpec((CIN - D, HID), full),
            pl.BlockSpec((1, HID), full),
            pl.BlockSpec((HID, D), full),
            pl.BlockSpec((1, D), full),
        ],
        out_specs=pl.BlockSpec((1, 1), full),
        out_shape=jax.ShapeDtypeStruct((1, 1), jnp.float32),
        scratch_shapes=[pltpu.VMEM((R, N), jnp.float32)],
    )(x_0, e_rand, aux.reshape(B, 1, 256), w1x, w1m, w1c2, w1c4,
      b1.reshape(1, HID), W2, b2.reshape(1, D))
    return out[0, 0] / jnp.float32(B * N * D)


# speedup vs baseline: 5.9632x; 5.9632x over previous
import functools

import jax
import jax.numpy as jnp
from jax.experimental import pallas as pl
from jax.experimental.pallas import tpu as pltpu

NUM_STEPS = 100
K = 16
BIG = 1e30
SELF_BIG = 2e30


def _fused_kernel(x_ref, e_ref, aux_ref, w1x_ref, w1m_ref, w1c2_ref, w1c4_ref,
                  b1_ref, w2_ref, b2_ref, out_ref, scr_ref, *, R, N):
    b = pl.program_id(0)
    rb = pl.program_id(1)

    @pl.when(jnp.logical_and(b == 0, rb == 0))
    def _init():
        out_ref[...] = jnp.zeros((1, 1), jnp.float32)

    x_all = x_ref[0]
    e_all = e_ref[0]
    c0 = aux_ref[0, 0, 128]
    c1 = aux_ref[0, 0, 129]

    xr = x_ref[0, pl.ds(rb * R, R), :]
    dx = xr[:, 0:1] - x_all[:, 0].reshape(1, N)
    dy = xr[:, 1:2] - x_all[:, 1].reshape(1, N)
    d2 = dx * dx + dy * dy

    col = jax.lax.broadcasted_iota(jnp.int32, (R, N), 1)
    row = jax.lax.broadcasted_iota(jnp.int32, (R, N), 0) + rb * R
    bits = jax.lax.bitcast_convert_type(d2, jnp.uint32)
    packed_bits = (bits & jnp.uint32(0xFFFFF800)) | col.astype(jnp.uint32)
    packed = jax.lax.bitcast_convert_type(packed_bits, jnp.float32)
    scr_ref[...] = jnp.where(col == row, SELF_BIG, packed)

    xn_all = c0 * x_all + c1 * e_all

    def body(i, _):
        p = scr_ref[...]
        for _ in range(4):
            m = jnp.min(p, axis=1, keepdims=True)
            p = jnp.where(p == m, BIG, p)
        scr_ref[...] = p
        return 0

    jax.lax.fori_loop(0, K // 4, body, 0)
    sel = (scr_ref[...] == BIG).astype(jnp.float32)
    acc = jax.lax.dot_general(sel, xn_all, (((1,), (0,)), ((), ())),
                              preferred_element_type=jnp.float32)
    xm = acc * jnp.float32(1.0 / K)

    ctxfeat = aux_ref[0, :, 0:128]
    w1c = w1c2_ref[...] + w1c4_ref[...]
    bias = jax.lax.dot_general(ctxfeat, w1c, (((1,), (0,)), ((), ())),
                               preferred_element_type=jnp.float32) + b1_ref[...]

    e_r = e_ref[0, pl.ds(rb * R, R), :]
    xn_r = c0 * xr + c1 * e_r
    pre = (jax.lax.dot_general(xn_r, w1x_ref[...], (((1,), (0,)), ((), ())),
                               preferred_element_type=jnp.float32)
           + jax.lax.dot_general(xm, w1m_ref[...], (((1,), (0,)), ((), ())),
                                 preferred_element_type=jnp.float32)
           + bias)
    h = jnp.maximum(pre, 0.0)
    out = jax.lax.dot_general(h, w2_ref[...], (((1,), (0,)), ((), ())),
                              preferred_element_type=jnp.float32) + b2_ref[...]
    err = out - e_r
    out_ref[...] += jnp.sum(err * err, keepdims=True)


def kernel(x_0, context, t, k, e_rand, W1, b1, W2, b2):
    B, N, D = x_0.shape
    R = min(2048, N)
    NB = N // R

    betas = jnp.linspace(1e-4, 0.02, NUM_STEPS).astype(jnp.float32)
    alphas = 1.0 - betas
    alpha_bars = jnp.cumprod(alphas)
    alpha_bar = alpha_bars[t]
    beta = betas[t]
    c0 = jnp.sqrt(alpha_bar)
    c1 = jnp.sqrt(1.0 - alpha_bar)

    aux = jnp.concatenate([
        context,
        beta[:, None], jnp.sin(beta)[:, None], jnp.cos(beta)[:, None],
        c0[:, None], c1[:, None],
        jnp.zeros((B, 126), jnp.float32),
    ], axis=1)

    CIN = context.shape[1] + 3 + D
    w1x = W1[0:D]
    w1m = W1[CIN:CIN + D]
    w1c2 = W1[D:CIN]
    w1c4 = W1[CIN + D:2 * CIN]
    HID = W1.shape[1]

    grid = (B, NB)
    full = lambda bi, ri: (0, 0)
    out = pl.pallas_call(
        functools.partial(_fused_kernel, R=R, N=N),
        grid=grid,
        in_specs=[
            pl.BlockSpec((1, N, D), lambda bi, ri: (bi, 0, 0)),
            pl.BlockSpec((1, N, D), lambda bi, ri: (bi, 0, 0)),
            pl.BlockSpec((1, 1, 256), lambda bi, ri: (bi, 0, 0)),
            pl.BlockSpec((D, HID), full),
            pl.BlockSpec((D, HID), full),
            pl.BlockSpec((CIN - D, HID), full),
            pl.BlockSpec((CIN - D, HID), full),
            pl.BlockSpec((1, HID), full),
            pl.BlockSpec((HID, D), full),
            pl.BlockSpec((1, D), full),
        ],
        out_specs=pl.BlockSpec((1, 1), full),
        out_shape=jax.ShapeDtypeStruct((1, 1), jnp.float32),
        scratch_shapes=[pltpu.VMEM((R, N), jnp.float32)],
    )(x_0, e_rand, aux.reshape(B, 1, 256), w1x, w1m, w1c2, w1c4,
      b1.reshape(1, HID), W2, b2.reshape(1, D))
    return out[0, 0] / jnp.float32(B * N * D)


# speedup vs baseline: 6.0528x; 1.0150x over previous
import functools

import jax
import jax.numpy as jnp
from jax.experimental import pallas as pl
from jax.experimental.pallas import tpu as pltpu

NUM_STEPS = 100
K = 16
BIG = 1e30
SELF_BIG = 2e30


def _fused_kernel(x_ref, e_ref, aux_ref, w1x_ref, w1m_ref, w1c2_ref, w1c4_ref,
                  b1_ref, w2_ref, b2_ref, out_ref, scr_ref, *, R, N):
    b = pl.program_id(0)
    rb = pl.program_id(1)

    @pl.when(jnp.logical_and(b == 0, rb == 0))
    def _init():
        out_ref[...] = jnp.zeros((1, 1), jnp.float32)

    x_all = x_ref[0]
    e_all = e_ref[0]
    c0 = aux_ref[0, 0, 128]
    c1 = aux_ref[0, 0, 129]

    xr = x_ref[0, pl.ds(rb * R, R), :]
    dx = xr[:, 0:1] - x_all[:, 0].reshape(1, N)
    dy = xr[:, 1:2] - x_all[:, 1].reshape(1, N)
    d2 = dx * dx + dy * dy

    col = jax.lax.broadcasted_iota(jnp.int32, (R, N), 1)
    row = jax.lax.broadcasted_iota(jnp.int32, (R, N), 0) + rb * R
    bits = jax.lax.bitcast_convert_type(d2, jnp.uint32)
    packed_bits = (bits & jnp.uint32(0xFFFFF800)) | col.astype(jnp.uint32)
    packed = jax.lax.bitcast_convert_type(packed_bits, jnp.float32)
    scr_ref[...] = jnp.where(col == row, SELF_BIG, packed)

    xn_all = c0 * x_all + c1 * e_all

    def body(i, _):
        p = scr_ref[...]
        for _ in range(8):
            m = jnp.min(p, axis=1, keepdims=True)
            p = jnp.where(p == m, BIG, p)
        scr_ref[...] = p
        return 0

    jax.lax.fori_loop(0, K // 8, body, 0)
    sel = (scr_ref[...] == BIG).astype(jnp.float32)
    acc = jax.lax.dot_general(sel, xn_all, (((1,), (0,)), ((), ())),
                              preferred_element_type=jnp.float32)
    xm = acc * jnp.float32(1.0 / K)

    ctxfeat = aux_ref[0, :, 0:128]
    w1c = w1c2_ref[...] + w1c4_ref[...]
    bias = jax.lax.dot_general(ctxfeat, w1c, (((1,), (0,)), ((), ())),
                               preferred_element_type=jnp.float32) + b1_ref[...]

    e_r = e_ref[0, pl.ds(rb * R, R), :]
    xn_r = c0 * xr + c1 * e_r
    pre = (jax.lax.dot_general(xn_r, w1x_ref[...], (((1,), (0,)), ((), ())),
                               preferred_element_type=jnp.float32)
           + jax.lax.dot_general(xm, w1m_ref[...], (((1,), (0,)), ((), ())),
                                 preferred_element_type=jnp.float32)
           + bias)
    h = jnp.maximum(pre, 0.0)
    out = jax.lax.dot_general(h, w2_ref[...], (((1,), (0,)), ((), ())),
                              preferred_element_type=jnp.float32) + b2_ref[...]
    err = out - e_r
    out_ref[...] += jnp.sum(err * err, keepdims=True)


def kernel(x_0, context, t, k, e_rand, W1, b1, W2, b2):
    B, N, D = x_0.shape
    R = min(2048, N)
    NB = N // R

    betas = jnp.linspace(1e-4, 0.02, NUM_STEPS).astype(jnp.float32)
    alphas = 1.0 - betas
    alpha_bars = jnp.cumprod(alphas)
    alpha_bar = alpha_bars[t]
    beta = betas[t]
    c0 = jnp.sqrt(alpha_bar)
    c1 = jnp.sqrt(1.0 - alpha_bar)

    aux = jnp.concatenate([
        context,
        beta[:, None], jnp.sin(beta)[:, None], jnp.cos(beta)[:, None],
        c0[:, None], c1[:, None],
        jnp.zeros((B, 126), jnp.float32),
    ], axis=1)

    CIN = context.shape[1] + 3 + D
    w1x = W1[0:D]
    w1m = W1[CIN:CIN + D]
    w1c2 = W1[D:CIN]
    w1c4 = W1[CIN + D:2 * CIN]
    HID = W1.shape[1]

    grid = (B, NB)
    full = lambda bi, ri: (0, 0)
    out = pl.pallas_call(
        functools.partial(_fused_kernel, R=R, N=N),
        grid=grid,
        in_specs=[
            pl.BlockSpec((1, N, D), lambda bi, ri: (bi, 0, 0)),
            pl.BlockSpec((1, N, D), lambda bi, ri: (bi, 0, 0)),
            pl.BlockSpec((1, 1, 256), lambda bi, ri: (bi, 0, 0)),
            pl.BlockSpec((D, HID), full),
            pl.BlockSpec((D, HID), full),
            pl.BlockSpec((CIN - D, HID), full),
            pl.BlockSpec((CIN - D, HID), full),
            pl.BlockSpec((1, HID), full),
            pl.BlockSpec((HID, D), full),
            pl.BlockSpec((1, D), full),
        ],
        out_specs=pl.BlockSpec((1, 1), full),
        out_shape=jax.ShapeDtypeStruct((1, 1), jnp.float32),
        scratch_shapes=[pltpu.VMEM((R, N), jnp.float32)],
    )(x_0, e_rand, aux.reshape(B, 1, 256), w1x, w1m, w1c2, w1c4,
      b1.reshape(1, HID), W2, b2.reshape(1, D))
    return out[0, 0] / jnp.float32(B * N * D)
